# baseline (speedup 1.0000x reference)
import functools

import jax
import jax.numpy as jnp
from jax import lax
from jax.experimental import pallas as pl
from jax.experimental.pallas import tpu as pltpu
from jax.experimental.pallas import tpu_sc as plsc

B = 5000
FANOUT = 10
D_IN = 128
D_H = 256
S = 40

G = B * FANOUT
C = 40
RC = C * FANOUT
NCHUNK = G // C
NW = 32


def _sc_mean_body(x2_hbm, m2_hbm, in_buf, out_buf):
    wid = lax.axis_index("s") * 2 + lax.axis_index("c")
    n_my = (NCHUNK - wid + NW - 1) // NW

    def chunk_body(i, carry):
        chunk = wid + i * NW
        pltpu.sync_copy(x2_hbm.at[pl.ds(chunk * RC, RC)], in_buf)

        def g_body(g, carry2):
            row0 = g * FANOUT
            for j in range(D_IN // 16):
                acc = in_buf[row0, pl.ds(j * 16, 16)]
                for r in range(1, FANOUT):
                    acc = acc + in_buf[row0 + r, pl.ds(j * 16, 16)]
                out_buf[g, pl.ds(j * 16, 16)] = acc * (1.0 / FANOUT)
            return carry2

        lax.fori_loop(0, C, g_body, 0)
        pltpu.sync_copy(out_buf, m2_hbm.at[pl.ds(chunk * C, C)])
        return carry

    lax.fori_loop(0, n_my, chunk_body, 0)


_sc_mean = functools.partial(
    pl.kernel,
    mesh=plsc.VectorSubcoreMesh(core_axis_name="c", subcore_axis_name="s"),
    out_type=jax.ShapeDtypeStruct((G, D_IN), jnp.float32),
    scratch_types=[
        pltpu.VMEM((RC, D_IN), jnp.float32),
        pltpu.VMEM((C, D_IN), jnp.float32),
    ],
)(_sc_mean_body)


def _tc_body(x0_ref, x1_ref, m2_ref, wa0_ref, ba0_ref, ws0_ref,
             wa1_ref, ba1_ref, ws1_ref, out_ref):
    wa0 = wa0_ref[...]
    ws0 = ws0_ref[...]
    ba0 = ba0_ref[...]

    x1 = x1_ref[...]
    m2 = m2_ref[...]
    h1 = (jnp.dot(x1, ws0, preferred_element_type=jnp.float32)
          + jnp.dot(m2, wa0, preferred_element_type=jnp.float32) + ba0)
    h1 = jnp.where(h1 >= 0, h1, 0.01 * h1)

    mh1 = h1.reshape(S, FANOUT, D_H).sum(axis=1) * (1.0 / FANOUT)
    m1 = x1.reshape(S, FANOUT, D_IN).sum(axis=1) * (1.0 / FANOUT)
    x0 = x0_ref[...]
    h0 = (jnp.dot(x0, ws0, preferred_element_type=jnp.float32)
          + jnp.dot(m1, wa0, preferred_element_type=jnp.float32) + ba0)
    h0 = jnp.where(h0 >= 0, h0, 0.01 * h0)

    out_ref[...] = (jnp.dot(h0, ws1_ref[...], preferred_element_type=jnp.float32)
                    + jnp.dot(mh1, wa1_ref[...], preferred_element_type=jnp.float32)
                    + ba1_ref[...])


def _tc_dense(x0, x1, m2, W_agg0, b_agg0, W_self0, W_agg1, b_agg1, W_self1):
    full = lambda shape: pl.BlockSpec(shape, lambda i: (0,) * len(shape))
    return pl.pallas_call(
        _tc_body,
        grid=(B // S,),
        in_specs=[
            pl.BlockSpec((S, D_IN), lambda i: (i, 0)),
            pl.BlockSpec((S * FANOUT, D_IN), lambda i: (i, 0)),
            pl.BlockSpec((S * FANOUT, D_IN), lambda i: (i, 0)),
            full((D_IN, D_H)),
            full((1, D_H)),
            full((D_IN, D_H)),
            full((D_H, D_H)),
            full((1, D_H)),
            full((D_H, D_H)),
        ],
        out_specs=pl.BlockSpec((S, D_H), lambda i: (i, 0)),
        out_shape=jax.ShapeDtypeStruct((B, D_H), jnp.float32),
        compiler_params=pltpu.CompilerParams(
            dimension_semantics=("arbitrary",),
        ),
    )(x0, x1, m2, W_agg0, b_agg0.reshape(1, D_H), W_self0,
      W_agg1, b_agg1.reshape(1, D_H), W_self1)


@jax.jit
def kernel(x0, x1, x2, W_agg0, b_agg0, W_self0, W_agg1, b_agg1, W_self1):
    m2 = _sc_mean(x2)
    return _tc_dense(x0, x1, m2, W_agg0, b_agg0, W_self0,
                     W_agg1, b_agg1, W_self1)


# speedup vs baseline: 1.8839x; 1.8839x over previous
import functools

import jax
import jax.numpy as jnp
from jax.experimental import pallas as pl
from jax.experimental.pallas import tpu as pltpu

B = 5000
FANOUT = 10
D_IN = 128
D_H = 256
S = 40

BF = jnp.bfloat16


def _dot(a, b):
    return jnp.dot(a, b, preferred_element_type=jnp.float32)


def _gcn_body(x0_ref, x1_ref, x2_ref, wa0_ref, ba0_ref, ws0_ref,
              wa1_ref, ba1_ref, ws1_ref, out_ref):
    ba0 = ba0_ref[...]

    x2 = x2_ref[...]
    m2 = x2.reshape(S * FANOUT, FANOUT, D_IN).sum(axis=1)
    x1b = x1_ref[...].astype(BF)
    h1 = _dot(x1b, ws0_ref[...]) + _dot(m2.astype(BF), wa0_ref[...]) + ba0
    h1 = jnp.where(h1 >= 0, h1, 0.01 * h1)

    mh1 = h1.reshape(S, FANOUT, D_H).sum(axis=1)
    m1 = x1_ref[...].reshape(S, FANOUT, D_IN).sum(axis=1)
    x0b = x0_ref[...].astype(BF)
    h0 = _dot(x0b, ws0_ref[...]) + _dot(m1.astype(BF), wa0_ref[...]) + ba0
    h0 = jnp.where(h0 >= 0, h0, 0.01 * h0)

    out_ref[...] = (_dot(h0.astype(BF), ws1_ref[...])
                    + _dot(mh1.astype(BF), wa1_ref[...]) + ba1_ref[...])


@jax.jit
def kernel(x0, x1, x2, W_agg0, b_agg0, W_self0, W_agg1, b_agg1, W_self1):
    scale = jnp.float32(1.0 / FANOUT)
    grid = (B // S,)
    full = lambda shape: pl.BlockSpec(shape, lambda i: (0,) * len(shape))
    return pl.pallas_call(
        _gcn_body,
        grid=grid,
        in_specs=[
            pl.BlockSpec((S, D_IN), lambda i: (i, 0)),
            pl.BlockSpec((S * FANOUT, D_IN), lambda i: (i, 0)),
            pl.BlockSpec((S * FANOUT * FANOUT, D_IN), lambda i: (i, 0)),
            full((D_IN, D_H)),
            full((1, D_H)),
            full((D_IN, D_H)),
            full((D_H, D_H)),
            full((1, D_H)),
            full((D_H, D_H)),
        ],
        out_specs=pl.BlockSpec((S, D_H), lambda i: (i, 0)),
        out_shape=jax.ShapeDtypeStruct((B, D_H), jnp.float32),
        compiler_params=pltpu.CompilerParams(
            dimension_semantics=("arbitrary",),
        ),
    )(x0, x1, x2, (W_agg0 * scale).astype(BF), b_agg0.reshape(1, D_H),
      W_self0.astype(BF), (W_agg1 * scale).astype(BF),
      b_agg1.reshape(1, D_H), W_self1.astype(BF))
